# baseline (speedup 1.0000x reference)
import functools

import jax
import jax.numpy as jnp
from jax import lax
from jax.experimental import pallas as pl
from jax.experimental.pallas import tpu as pltpu
from jax.experimental.pallas import tpu_sc as plsc

_LANES = 16


@functools.lru_cache(maxsize=None)
def _build_sc_kernel(n_tokens, n_morph, n_ptab, emb_dim):
    info = plsc.get_sparse_core_info()
    num_cores, num_subcores = info.num_cores, info.num_subcores
    n_workers = num_cores * num_subcores
    tok_per_worker = n_tokens // n_workers
    C = 128
    n_chunks = tok_per_worker // C
    n_col = emb_dim // _LANES
    n_morph2 = 2 * n_morph
    n_grp = n_morph // _LANES
    inv128 = 1.0 / (4 * n_morph)
    inv4 = 0.25

    mesh = plsc.VectorSubcoreMesh(core_axis_name="c", subcore_axis_name="s")

    @functools.partial(
        pl.kernel,
        out_type=jax.ShapeDtypeStruct((n_tokens, emb_dim), jnp.float32),
        mesh=mesh,
        compiler_params=pltpu.CompilerParams(use_tc_tiling_on_sc=False,
                                             needs_layout_passes=False),
        scratch_types=[
            pltpu.VMEM((C, n_morph), jnp.int32),
            pltpu.VMEM((C, n_morph), jnp.int32),
            pltpu.VMEM((C, n_morph), jnp.int32),
            pltpu.VMEM((4, C), jnp.int32),
            pltpu.VMEM((4, n_morph2, C), jnp.int32),
            pltpu.VMEM((n_ptab, emb_dim), jnp.float32),
            pltpu.VMEM((tok_per_worker, emb_dim), jnp.float32),
            pltpu.VMEM((tok_per_worker, emb_dim), jnp.float32),
            pltpu.VMEM((C, emb_dim), jnp.float32),
            pltpu.SemaphoreType.DMA,
            pltpu.SemaphoreType.DMA,
        ],
    )
    def sc_kernel(fidx_hbm, lidx_hbm, pidx_hbm, widx_hbm,
                  word_hbm, postag_hbm, out_hbm,
                  fm_v, lm_v, pm_v, wi_v, smi_v, ptab_v,
                  acc_v, wrow_v, outb_v, sem_i, sem_a):
        wid = lax.axis_index("s") * num_cores + lax.axis_index("c")
        base = wid * tok_per_worker

        pltpu.sync_copy(postag_hbm, ptab_v)
        lane = lax.iota(jnp.int32, _LANES)

        wrow_cps = []
        for ci in range(n_chunks):
            tok = ci * C
            pltpu.sync_copy(fidx_hbm.at[pl.ds(base + tok, C)], fm_v)
            pltpu.sync_copy(lidx_hbm.at[pl.ds(base + tok, C)], lm_v)
            pltpu.sync_copy(pidx_hbm.at[pl.ds(base + tok, C)], pm_v)
            pltpu.sync_copy(widx_hbm.at[pl.ds(base + tok, C)], wi_v.at[ci])

            def prep_body(t, _):
                col = jnp.full((_LANES,), t, jnp.int32)
                for g in range(n_grp):
                    sl = pl.ds(g * _LANES, _LANES)
                    plsc.store_scatter(
                        smi_v.at[ci], [lane + g * _LANES, col], fm_v[t, sl])
                    plsc.store_scatter(
                        smi_v.at[ci], [lane + (n_morph + g * _LANES), col],
                        lm_v[t, sl])
                pvecs = [pm_v[t, pl.ds(g * _LANES, _LANES)]
                         for g in range(n_grp)]
                accs = [jnp.zeros((_LANES,), jnp.float32)] * n_col
                for r in range(n_morph):
                    p = pvecs[r // _LANES][r % _LANES]
                    accs = [accs[c] + ptab_v[p, pl.ds(c * _LANES, _LANES)]
                            for c in range(n_col)]
                for c in range(n_col):
                    acc_v[tok + t, pl.ds(c * _LANES, _LANES)] = accs[c]
                return 0

            lax.fori_loop(0, C, prep_body, 0)

            wrow_cps.append(
                pltpu.async_copy(word_hbm.at[wi_v.at[ci]],
                                 wrow_v.at[pl.ds(tok, C)], sem_i))

            def morph_body(r, _):
                pltpu.async_copy(word_hbm.at[smi_v.at[ci, r]],
                                 acc_v.at[pl.ds(tok, C)], sem_a, add=True)
                return 0

            lax.fori_loop(0, n_morph2, morph_body, 0)

        def drain_body(_, __):
            pltpu.make_async_copy(word_hbm.at[smi_v.at[0, 0]],
                                  acc_v.at[pl.ds(0, C)], sem_a).wait()
            return 0

        lax.fori_loop(0, n_chunks * n_morph2, drain_body, 0)
        for cp in wrow_cps:
            cp.wait()

        for ci in range(n_chunks):
            tok = ci * C

            def out_body(t, _):
                for c in range(n_col):
                    sl = pl.ds(c * _LANES, _LANES)
                    outb_v[t, sl] = (acc_v[tok + t, sl] * inv128
                                     + wrow_v[tok + t, sl] * inv4)
                return 0

            lax.fori_loop(0, C, out_body, 0)
            pltpu.sync_copy(outb_v, out_hbm.at[pl.ds(base + tok, C)])

    return sc_kernel


def kernel(word_idx, form_idx, lemma_idx, postag_idx, word_table, postag_table):
    n = word_idx.shape[0]
    n_morph = form_idx.shape[1] * form_idx.shape[2]
    emb_dim = word_table.shape[1]

    sc = _build_sc_kernel(n, n_morph, postag_table.shape[0], emb_dim)
    return sc(form_idx.reshape(n, n_morph).astype(jnp.int32),
              lemma_idx.reshape(n, n_morph).astype(jnp.int32),
              postag_idx.reshape(n, n_morph).astype(jnp.int32),
              word_idx.astype(jnp.int32),
              word_table.astype(jnp.float32),
              postag_table.astype(jnp.float32))


# speedup vs baseline: 1.0383x; 1.0383x over previous
import functools

import jax
import jax.numpy as jnp
from jax import lax
from jax.experimental import pallas as pl
from jax.experimental.pallas import tpu as pltpu
from jax.experimental.pallas import tpu_sc as plsc

_LANES = 16


def _tc_prep(form, lemma, word2d, n_blocks, n_morph, C):
    n_rows = 8 * ((2 * n_morph + 1 + 7) // 8)

    def body(f_ref, l_ref, w_ref, o_ref):
        o_ref[0, 0:n_morph, :] = jnp.transpose(f_ref[...], (1, 0))
        o_ref[0, n_morph:2 * n_morph, :] = jnp.transpose(l_ref[...], (1, 0))
        o_ref[0, 2 * n_morph:n_rows, :] = jnp.broadcast_to(
            w_ref[0], (n_rows - 2 * n_morph, C))

    return pl.pallas_call(
        body,
        grid=(n_blocks,),
        in_specs=[
            pl.BlockSpec((C, n_morph), lambda b: (b, 0)),
            pl.BlockSpec((C, n_morph), lambda b: (b, 0)),
            pl.BlockSpec((1, 1, C), lambda b: (b, 0, 0)),
        ],
        out_specs=pl.BlockSpec((1, n_rows, C), lambda b: (b, 0, 0)),
        out_shape=jax.ShapeDtypeStruct((n_blocks, n_rows, C), jnp.int32),
    )(form, lemma, word2d)


@functools.lru_cache(maxsize=None)
def _build_sc_kernel(n_tokens, n_rows, n_morph, n_ptab, emb_dim):
    info = plsc.get_sparse_core_info()
    num_cores, num_subcores = info.num_cores, info.num_subcores
    n_workers = num_cores * num_subcores
    tok_per_worker = n_tokens // n_workers
    C = 128
    n_chunks = tok_per_worker // C
    n_col = emb_dim // _LANES
    n_morph2 = 2 * n_morph
    n_grp = n_morph // _LANES
    inv128 = 1.0 / (4 * n_morph)
    inv4 = 0.25

    mesh = plsc.VectorSubcoreMesh(core_axis_name="c", subcore_axis_name="s")

    @functools.partial(
        pl.kernel,
        out_type=jax.ShapeDtypeStruct((n_tokens, emb_dim), jnp.float32),
        mesh=mesh,
        compiler_params=pltpu.CompilerParams(use_tc_tiling_on_sc=False),
        scratch_types=[
            pltpu.VMEM((2, n_rows, C), jnp.int32),
            pltpu.VMEM((C, n_morph), jnp.int32),
            pltpu.VMEM((n_ptab, emb_dim), jnp.float32),
            pltpu.VMEM((tok_per_worker, emb_dim), jnp.float32),
            pltpu.VMEM((tok_per_worker, emb_dim), jnp.float32),
            pltpu.VMEM((C, emb_dim), jnp.float32),
            pltpu.SemaphoreType.DMA,
            pltpu.SemaphoreType.DMA,
        ],
    )
    def sc_kernel(idx_hbm, pidx_hbm, word_hbm, postag_hbm, out_hbm,
                  idx_v, pidx_v, ptab_v, acc_v, wrow_v, outb_v, sem_i, sem_a):
        wid = lax.axis_index("s") * num_cores + lax.axis_index("c")
        base = wid * tok_per_worker
        blk0 = wid * n_chunks

        pltpu.sync_copy(postag_hbm, ptab_v)

        wrow_cps = []
        for ci in range(n_chunks):
            buf = ci % 2
            pltpu.sync_copy(idx_hbm.at[blk0 + ci], idx_v.at[buf])
            pltpu.sync_copy(pidx_hbm.at[pl.ds(base + ci * C, C)], pidx_v)
            tok = ci * C

            def pos_body(t, _):
                pvecs = [pidx_v[t, pl.ds(g * _LANES, _LANES)]
                         for g in range(n_grp)]
                accs = [jnp.zeros((_LANES,), jnp.float32)] * n_col
                for r in range(n_morph):
                    p = pvecs[r // _LANES][r % _LANES]
                    accs = [accs[c] + ptab_v[p, pl.ds(c * _LANES, _LANES)]
                            for c in range(n_col)]
                for c in range(n_col):
                    acc_v[tok + t, pl.ds(c * _LANES, _LANES)] = accs[c]
                return 0

            lax.fori_loop(0, C, pos_body, 0)

            wrow_cps.append(
                pltpu.async_copy(word_hbm.at[idx_v.at[buf, n_morph2]],
                                 wrow_v.at[pl.ds(tok, C)], sem_i))

            def morph_body(r, _):
                pltpu.async_copy(word_hbm.at[idx_v.at[buf, r]],
                                 acc_v.at[pl.ds(tok, C)], sem_a, add=True)
                return 0

            lax.fori_loop(0, n_morph2, morph_body, 0)

        def drain_body(_, __):
            pltpu.make_async_copy(word_hbm.at[idx_v.at[0, 0]],
                                  acc_v.at[pl.ds(0, C)], sem_a).wait()
            return 0

        lax.fori_loop(0, n_chunks * n_morph2, drain_body, 0)
        for cp in wrow_cps:
            cp.wait()

        for ci in range(n_chunks):
            tok = ci * C

            def out_body(t, _):
                for c in range(n_col):
                    sl = pl.ds(c * _LANES, _LANES)
                    outb_v[t, sl] = (acc_v[tok + t, sl] * inv128
                                     + wrow_v[tok + t, sl] * inv4)
                return 0

            lax.fori_loop(0, C, out_body, 0)
            pltpu.sync_copy(outb_v, out_hbm.at[pl.ds(base + tok, C)])

    return sc_kernel


def kernel(word_idx, form_idx, lemma_idx, postag_idx, word_table, postag_table):
    n = word_idx.shape[0]
    n_morph = form_idx.shape[1] * form_idx.shape[2]
    emb_dim = word_table.shape[1]
    C = 128
    n_blocks = n // C
    n_rows = 8 * ((2 * n_morph + 1 + 7) // 8)

    blocks = _tc_prep(form_idx.reshape(n, n_morph).astype(jnp.int32),
                      lemma_idx.reshape(n, n_morph).astype(jnp.int32),
                      word_idx.astype(jnp.int32).reshape(n_blocks, 1, C),
                      n_blocks, n_morph, C)
    pidx = postag_idx.reshape(n, n_morph).astype(jnp.int32)

    sc = _build_sc_kernel(n, n_rows, n_morph, postag_table.shape[0], emb_dim)
    return sc(blocks, pidx, word_table.astype(jnp.float32),
              postag_table.astype(jnp.float32))
